# baseline (speedup 1.0000x reference)
import functools

import jax
import jax.numpy as jnp
from jax import lax
from jax.experimental import pallas as pl
from jax.experimental.pallas import tpu as pltpu
from jax.experimental.pallas import tpu_sc as plsc

_NCELL = 8
_FH, _FW = 3, 5
_S = 14
_WIN = _FH * _FW
_W = 224


def _hist_body(xx_hbm, out_hbm, buf, stage):
    c = lax.axis_index("c")
    s = lax.axis_index("s")
    wid = c * 16 + s
    i = wid // 2
    half = wid % 2
    row0 = _S * i
    arow = 8 * (row0 // 8)
    roff = row0 - arow

    pltpu.sync_copy(xx_hbm.at[pl.ds(arow, 16), :], buf)

    lane = lax.iota(jnp.int32, 16)
    dy = jnp.where(lane < _FW, 0, jnp.where(lane < 2 * _FW, 1, 2))
    dx = lane - _FW * dy
    valid = lane < _WIN
    rowidx = dy + roff
    colbase = dx + (_NCELL * _S) * half

    out = jnp.zeros((16,), jnp.float32)
    for jl in range(_NCELL):
        vals = plsc.load_gather(buf, [rowidx, colbase + _S * jl], mask=valid)
        pred = jnp.logical_and(vals <= 0.0, valid)
        cnt = plsc.all_reduce_population_count(pred).astype(jnp.float32)
        out = jnp.where(lane == 2 * jl, cnt, out)
        out = jnp.where(lane == 2 * jl + 1, float(_WIN) - cnt, out)

    stage[...] = out
    pltpu.sync_copy(stage, out_hbm.at[pl.ds(wid * 16, 16)])


@functools.cache
def _hist_sc():
    return functools.partial(
        pl.kernel,
        out_type=jax.ShapeDtypeStruct((512,), jnp.float32),
        mesh=plsc.VectorSubcoreMesh(core_axis_name="c", subcore_axis_name="s"),
        compiler_params=pltpu.CompilerParams(needs_layout_passes=False),
        scratch_types=[
            pltpu.VMEM((16, _W), jnp.float32),
            pltpu.VMEM((16,), jnp.float32),
        ],
    )(_hist_body)


def kernel(xx):
    return _hist_sc()(xx).reshape(16, 16, 2)


# speedup vs baseline: 1.0603x; 1.0603x over previous
import functools

import jax
import jax.numpy as jnp
from jax import lax
from jax.experimental import pallas as pl
from jax.experimental.pallas import tpu as pltpu
from jax.experimental.pallas import tpu_sc as plsc

_FH, _FW = 3, 5
_S = 14
_WIN = _FH * _FW
_W = 224


def _hist_body(xx_hbm, out_hbm, buf, stage):
    i = lax.axis_index("s")
    row0 = _S * i
    arow = 8 * (row0 // 8)
    roff = row0 - arow

    pltpu.sync_copy(xx_hbm.at[pl.ds(arow, 16), :], buf)

    lane = lax.iota(jnp.int32, 16)
    dy = jnp.where(lane < _FW, 0, jnp.where(lane < 2 * _FW, 1, 2))
    dx = lane - _FW * dy
    valid = lane < _WIN
    rowidx = dy + roff

    for h in range(2):
        out = jnp.zeros((16,), jnp.float32)
        for k in range(8):
            jl = 8 * h + k
            vals = plsc.load_gather(buf, [rowidx, dx + _S * jl], mask=valid)
            pred = jnp.logical_and(vals <= 0.0, valid)
            cnt = plsc.all_reduce_population_count(pred).astype(jnp.float32)
            out = jnp.where(lane == 2 * k, cnt, out)
            out = jnp.where(lane == 2 * k + 1, float(_WIN) - cnt, out)
        stage[pl.ds(16 * h, 16)] = out

    pltpu.sync_copy(stage, out_hbm.at[pl.ds(i * 32, 32)])


@functools.cache
def _hist_sc():
    return functools.partial(
        pl.kernel,
        out_type=jax.ShapeDtypeStruct((512,), jnp.float32),
        mesh=plsc.VectorSubcoreMesh(
            core_axis_name="c", subcore_axis_name="s", num_cores=1
        ),
        compiler_params=pltpu.CompilerParams(needs_layout_passes=False),
        scratch_types=[
            pltpu.VMEM((16, _W), jnp.float32),
            pltpu.VMEM((32,), jnp.float32),
        ],
    )(_hist_body)


def kernel(xx):
    return _hist_sc()(xx).reshape(16, 16, 2)


# speedup vs baseline: 1.0710x; 1.0101x over previous
import functools

import jax
import jax.numpy as jnp
from jax import lax
from jax.experimental import pallas as pl
from jax.experimental.pallas import tpu as pltpu
from jax.experimental.pallas import tpu_sc as plsc

_FH, _FW = 3, 5
_S = 14
_WIN = _FH * _FW
_W = 224


def _hist_body(xx_hbm, out_hbm, buf, stage):
    i = lax.axis_index("s")
    row0 = _S * i
    arow = 8 * (row0 // 8)
    roff = row0 - arow

    pltpu.sync_copy(xx_hbm.at[pl.ds(arow, 16), :], buf)

    lane = lax.iota(jnp.int32, 16)
    cols = _S * lane
    acc = jnp.zeros((16,), jnp.float32)
    for dy in range(_FH):
        rowv = jnp.broadcast_to(roff + dy, (16,)).astype(jnp.int32)
        for dx in range(_FW):
            vals = plsc.load_gather(buf, [rowv, cols + dx])
            acc = acc + jnp.where(vals <= 0.0, 1.0, 0.0)

    plsc.store_scatter(stage, [2 * lane], acc)
    plsc.store_scatter(stage, [2 * lane + 1], float(_WIN) - acc)

    pltpu.sync_copy(stage, out_hbm.at[pl.ds(i * 32, 32)])


@functools.cache
def _hist_sc():
    return functools.partial(
        pl.kernel,
        out_type=jax.ShapeDtypeStruct((512,), jnp.float32),
        mesh=plsc.VectorSubcoreMesh(
            core_axis_name="c", subcore_axis_name="s", num_cores=1
        ),
        compiler_params=pltpu.CompilerParams(needs_layout_passes=False),
        scratch_types=[
            pltpu.VMEM((16, _W), jnp.float32),
            pltpu.VMEM((32,), jnp.float32),
        ],
    )(_hist_body)


def kernel(xx):
    return _hist_sc()(xx).reshape(16, 16, 2)


# speedup vs baseline: 1.0868x; 1.0148x over previous
import functools

import jax
import jax.numpy as jnp
from jax import lax
from jax.experimental import pallas as pl
from jax.experimental.pallas import tpu as pltpu
from jax.experimental.pallas import tpu_sc as plsc

_FH, _FW = 3, 5
_S = 14
_WIN = _FH * _FW
_W = 224
_SPAN = 672


def _body(xx_hbm, out_hbm, buf, stage):
    i = lax.axis_index("s")
    base = (_S * i) * _W
    pltpu.sync_copy(xx_hbm.at[pl.ds(base, _SPAN)], buf)

    lane = lax.iota(jnp.int32, 16)
    cols = _S * lane
    acc = jnp.zeros((16,), jnp.float32)
    for dy in range(_FH):
        for dx in range(_FW):
            vals = plsc.load_gather(buf, [cols + (dy * _W + dx)])
            acc = acc + jnp.where(vals <= 0.0, 1.0, 0.0)

    plsc.store_scatter(stage, [2 * lane], acc)
    plsc.store_scatter(stage, [2 * lane + 1], float(_WIN) - acc)
    pltpu.sync_copy(stage, out_hbm.at[pl.ds(i * 32, 32)])


@functools.cache
def _k():
    return functools.partial(
        pl.kernel,
        out_type=jax.ShapeDtypeStruct((512,), jnp.float32),
        mesh=plsc.VectorSubcoreMesh(
            core_axis_name="c", subcore_axis_name="s", num_cores=1
        ),
        compiler_params=pltpu.CompilerParams(needs_layout_passes=False, disable_bounds_checks=True, disable_semaphore_checks=True),
        scratch_types=[
            pltpu.VMEM((_SPAN,), jnp.float32),
            pltpu.VMEM((32,), jnp.float32),
        ],
    )(_body)


def kernel(xx):
    return _k()(xx.reshape(-1)).reshape(16, 16, 2)
